# baseline (speedup 1.0000x reference)
import jax
import jax.numpy as jnp
from jax.experimental import pallas as pl
from jax.experimental.pallas import tpu as pltpu

N = 100000
D = 128
B = 64
BN1 = 10000
BN2 = 10000
CH = 2000
NB1 = N // BN1
NB2 = N // BN2
INV_LOG_B = 0.240482983169996


def _fused_kernel(lk_ref, v_ref, wr_ref, lq_ref, h_ref,
                  out_ref, conf_ref, acc_ref, cnt_ref, p2_ref):
    i = pl.program_id(0)

    @pl.when(i < NB1)
    def _bucket():
        for k in range(BN1 // CH):
            sl = pl.ds(k * CH, CH)
            lk = lk_ref[sl, :]
            v = v_ref[sl, :]
            rowmax = jnp.max(lk, axis=-1, keepdims=True)
            ji = jax.lax.broadcasted_iota(jnp.int32, (CH, B), 1)
            idx = jnp.min(jnp.where(lk == rowmax, ji, B), axis=-1,
                          keepdims=True)
            m = jnp.where(ji == idx, 1.0, 0.0)
            dn = (((0,), (0,)), ((), ()))
            part = jax.lax.dot_general(m, v, dn,
                                       preferred_element_type=jnp.float32)
            partc = jnp.sum(m, axis=0, keepdims=True)

            @pl.when(jnp.logical_or(i != 0, k != 0))
            def _accum():
                acc_ref[...] += part
                cnt_ref[...] += jnp.broadcast_to(partc, (8, B))

            @pl.when(jnp.logical_and(i == 0, k == 0))
            def _init():
                acc_ref[...] = part
                cnt_ref[...] = jnp.broadcast_to(partc, (8, B))

    @pl.when(i == NB1)
    def _proto():
        cnt = jnp.transpose(cnt_ref[0:1, :], (1, 0))
        acc = acc_ref[...]
        vmean = jnp.sum(acc, axis=0, keepdims=True) * (1.0 / N)
        proto = jnp.where(cnt == 0.0, vmean, acc / jnp.clip(cnt, 1.0, None))
        dn = (((1,), (1,)), ((), ()))
        p2_ref[...] = jax.lax.dot_general(
            proto, wr_ref[...], dn, preferred_element_type=jnp.float32)

    @pl.when(i >= NB1)
    def _recover():
        for k in range(BN2 // CH):
            sl = pl.ds(k * CH, CH)
            lq = lq_ref[sl, :]
            mx = jnp.max(lq, axis=-1, keepdims=True)
            t = lq - mx
            e = jnp.exp(t)
            s = jnp.sum(e, axis=-1, keepdims=True)
            u = jnp.sum(e * t, axis=-1, keepdims=True)
            rs = 1.0 / s
            gate = (jnp.log(s) - u * rs) * INV_LOG_B
            pq = e * rs
            resid = jnp.dot(pq, p2_ref[...],
                            preferred_element_type=jnp.float32)
            out_ref[sl, :] = h_ref[sl, :] + gate * resid
            conf_ref[sl, :] = 1.0 - gate


@jax.jit
def kernel(h_fused, V, bucket_logits_q, bucket_logits_k, W_r):
    out, conf = pl.pallas_call(
        _fused_kernel,
        grid=(NB1 + NB2,),
        in_specs=[
            pl.BlockSpec((BN1, B), lambda i: (jnp.minimum(i, NB1 - 1), 0)),
            pl.BlockSpec((BN1, D), lambda i: (jnp.minimum(i, NB1 - 1), 0)),
            pl.BlockSpec((D, D), lambda i: (0, 0)),
            pl.BlockSpec((BN2, B), lambda i: (jnp.maximum(i - NB1, 0), 0)),
            pl.BlockSpec((BN2, D), lambda i: (jnp.maximum(i - NB1, 0), 0)),
        ],
        out_specs=[
            pl.BlockSpec((BN2, D), lambda i: (jnp.maximum(i - NB1, 0), 0)),
            pl.BlockSpec((BN2, 1), lambda i: (jnp.maximum(i - NB1, 0), 0)),
        ],
        out_shape=[
            jax.ShapeDtypeStruct((N, D), jnp.float32),
            jax.ShapeDtypeStruct((N, 1), jnp.float32),
        ],
        scratch_shapes=[
            pltpu.VMEM((B, D), jnp.float32),
            pltpu.VMEM((8, B), jnp.float32),
            pltpu.VMEM((B, D), jnp.float32),
        ],
        compiler_params=pltpu.CompilerParams(
            dimension_semantics=("arbitrary",),
            vmem_limit_bytes=63 * 1024 * 1024,
            allow_input_fusion=(True, True, True, True, True),
            fuse_transposed_lhs_in_matmul=True),
    )(bucket_logits_k, V, W_r, bucket_logits_q, h_fused)

    return (out, conf.reshape(N))


# speedup vs baseline: 1.0198x; 1.0198x over previous
import jax
import jax.numpy as jnp
from jax.experimental import pallas as pl
from jax.experimental.pallas import tpu as pltpu

N = 100000
D = 128
B = 64
BN1 = 10000
BN2 = 10000
CH = 2000
NB1 = N // BN1
NB2 = N // BN2
INV_LOG_B = 0.240482983169996


def _fused_kernel(lk_ref, v_ref, wr_ref, lq_ref, h_ref,
                  out_ref, conf_ref, acc_ref, cnt_ref, p2_ref):
    i = pl.program_id(0)

    @pl.when(i == 0)
    def _zero():
        acc_ref[...] = jnp.zeros((B, D), jnp.float32)
        cnt_ref[...] = jnp.zeros((8, B), jnp.float32)

    @pl.when(i < NB1)
    def _bucket():
        for k in range(BN1 // CH):
            sl = pl.ds(k * CH, CH)
            lk = lk_ref[sl, :]
            v = v_ref[sl, :]
            rowmax = jnp.max(lk, axis=-1, keepdims=True)
            ji = jax.lax.broadcasted_iota(jnp.int32, (CH, B), 1)
            idx = jnp.min(jnp.where(lk == rowmax, ji, B), axis=-1,
                          keepdims=True)
            m = jnp.where(ji == idx, 1.0, 0.0)
            dn = (((0,), (0,)), ((), ()))
            part = jax.lax.dot_general(m, v, dn,
                                       preferred_element_type=jnp.float32)
            partc = jnp.sum(m, axis=0, keepdims=True)
            acc_ref[...] += part
            cnt_ref[...] += jnp.broadcast_to(partc, (8, B))

    @pl.when(i == NB1)
    def _proto():
        cnt = jnp.transpose(cnt_ref[0:1, :], (1, 0))
        acc = acc_ref[...]
        vmean = jnp.sum(acc, axis=0, keepdims=True) * (1.0 / N)
        proto = jnp.where(cnt == 0.0, vmean, acc / jnp.clip(cnt, 1.0, None))
        dn = (((1,), (1,)), ((), ()))
        p2_ref[...] = jax.lax.dot_general(
            proto, wr_ref[...], dn, preferred_element_type=jnp.float32)

    @pl.when(i >= NB1)
    def _recover():
        for k in range(BN2 // CH):
            sl = pl.ds(k * CH, CH)
            lq = lq_ref[sl, :]
            mx = jnp.max(lq, axis=-1, keepdims=True)
            t = lq - mx
            e = jnp.exp(t)
            s = jnp.sum(e, axis=-1, keepdims=True)
            u = jnp.sum(e * t, axis=-1, keepdims=True)
            rs = 1.0 / s
            gate = (jnp.log(s) - u * rs) * INV_LOG_B
            pq = e * rs
            resid = jnp.dot(pq, p2_ref[...],
                            preferred_element_type=jnp.float32)
            out_ref[sl, :] = h_ref[sl, :] + gate * resid
            conf_ref[sl, :] = 1.0 - gate


@jax.jit
def kernel(h_fused, V, bucket_logits_q, bucket_logits_k, W_r):
    out, conf = pl.pallas_call(
        _fused_kernel,
        grid=(NB1 + NB2,),
        in_specs=[
            pl.BlockSpec((BN1, B), lambda i: (jnp.minimum(i, NB1 - 1), 0)),
            pl.BlockSpec((BN1, D), lambda i: (jnp.minimum(i, NB1 - 1), 0)),
            pl.BlockSpec((D, D), lambda i: (0, 0)),
            pl.BlockSpec((BN2, B), lambda i: (jnp.maximum(i - NB1, 0), 0)),
            pl.BlockSpec((BN2, D), lambda i: (jnp.maximum(i - NB1, 0), 0)),
        ],
        out_specs=[
            pl.BlockSpec((BN2, D), lambda i: (jnp.maximum(i - NB1, 0), 0)),
            pl.BlockSpec((BN2, 1), lambda i: (jnp.maximum(i - NB1, 0), 0)),
        ],
        out_shape=[
            jax.ShapeDtypeStruct((N, D), jnp.float32),
            jax.ShapeDtypeStruct((N, 1), jnp.float32),
        ],
        scratch_shapes=[
            pltpu.VMEM((B, D), jnp.float32),
            pltpu.VMEM((8, B), jnp.float32),
            pltpu.VMEM((B, D), jnp.float32),
        ],
        compiler_params=pltpu.CompilerParams(
            dimension_semantics=("arbitrary",),
            vmem_limit_bytes=63 * 1024 * 1024,
            allow_input_fusion=(True, True, True, True, True),
            fuse_transposed_lhs_in_matmul=True),
    )(bucket_logits_k, V, W_r, bucket_logits_q, h_fused)

    return (out, conf.reshape(N))
